# Initial kernel scaffold; baseline (speedup 1.0000x reference)
import jax
import jax.numpy as jnp
from jax.experimental import pallas as pl


def kernel(input_h, edges, W, a):
    raise NotImplementedError("write your pallas kernel here")


# speedup vs baseline: 2.9989x; 2.9989x over previous
import jax
import jax.numpy as jnp
from jax import lax
from jax.experimental import pallas as pl
from jax.experimental.pallas import tpu as pltpu
from jax.experimental.pallas import tpu_sc as plsc

H = 4
D = 256
HD = 128
N = 10000
E = 160000
LEAK = 0.01
NB = 10
BN = N // NB
NT = 16

EPT = E // NT

P = H * E
K = 80
PPT = P // NT
SP = 8000
NSUPER = PPT // SP
CPS = SP // K
RSTART = 624
RLEN = 640


def _dense_body(inp_ref, w_ref, a_ref, xsplit_ref, hsplit_ref, s_ref):
    inp = inp_ref[...]
    acc = jnp.zeros((BN, D), jnp.float32)
    s1s, s2s = [], []
    for h in range(H):
        wh = w_ref[h]
        xb = lax.dot_general(inp, wh, (((1,), (1,)), ((), ())),
                             preferred_element_type=jnp.float32)
        xsplit_ref[0, h] = xb[:, :HD]
        xsplit_ref[1, h] = xb[:, HD:]
        acc = acc + xb
        a1 = a_ref[h, 0, :D]
        a2 = a_ref[h, 0, D:]
        s1s.append(lax.dot_general(xb, a1, (((1,), (0,)), ((), ())),
                                   preferred_element_type=jnp.float32))
        s2s.append(lax.dot_general(xb, a2, (((1,), (0,)), ((), ())),
                                   preferred_element_type=jnp.float32))
    acc = acc * (1.0 / H)
    hsplit_ref[0] = acc[:, :HD]
    hsplit_ref[1] = acc[:, HD:]
    s_ref[...] = jnp.stack(s1s + s2s, axis=1)


def _dense(input_h, W, a):
    return pl.pallas_call(
        _dense_body,
        grid=(NB,),
        in_specs=[
            pl.BlockSpec((BN, D), lambda i: (i, 0)),
            pl.BlockSpec((H, D, D), lambda i: (0, 0, 0)),
            pl.BlockSpec((H, 1, 2 * D), lambda i: (0, 0, 0)),
        ],
        out_specs=[
            pl.BlockSpec((2, 1, BN, HD), lambda i: (0, 0, i, 0)),
            pl.BlockSpec((2, BN, HD), lambda i: (0, i, 0)),
            pl.BlockSpec((BN, 2 * H), lambda i: (i, 0)),
        ],
        out_shape=[
            jax.ShapeDtypeStruct((2, H, N, HD), jnp.float32),
            jax.ShapeDtypeStruct((2, N, HD), jnp.float32),
            jax.ShapeDtypeStruct((N, 2 * H), jnp.float32),
        ],
    )(input_h, W, a)


def _softmax_body(s2d, src_hbm, dst_hbm, att_hbm,
                  src_v, dst_v, s_v, exp_v, zstage_v, ztmp_v, zsh):
    c = lax.axis_index("c")
    t = lax.axis_index("s")
    base = t * EPT
    pltpu.sync_copy(src_hbm.at[pl.ds(base, EPT)], src_v)
    pltpu.sync_copy(dst_hbm.at[pl.ds(base, EPT)], dst_v)
    pltpu.sync_copy(s2d, s_v)

    sums = []
    for hh in range(2):
        h = 2 * c + hh
        hv1 = jnp.zeros((16,), jnp.int32) + h
        hv2 = hv1 + H

        def body(i, acc, hh=hh, hv1=hv1, hv2=hv2):
            sv = src_v[pl.ds(i * 16, 16)]
            dv = dst_v[pl.ds(i * 16, 16)]
            sc = (plsc.load_gather(s_v, [sv * (2 * H) + hv1])
                  + plsc.load_gather(s_v, [dv * (2 * H) + hv2]))
            sc = jnp.where(sc >= 0, sc, LEAK * sc)
            ev = jnp.exp(sc)
            exp_v[pl.ds(hh * EPT + i * 16, 16)] = ev
            return acc + ev

        acc = lax.fori_loop(0, EPT // 16, body, jnp.zeros((16,), jnp.float32))
        sums.append(jnp.sum(acc, axis=0))

    lane = lax.iota(jnp.int32, 16)
    ztile = jnp.where(lane == 0, sums[0], jnp.where(lane == 1, sums[1], 0.0))
    zstage_v[...] = ztile
    pltpu.sync_copy(zstage_v, zsh.at[pl.ds(t * 16, 16)])
    plsc.subcore_barrier()
    pltpu.sync_copy(zsh, ztmp_v)
    ztot = jnp.zeros((16,), jnp.float32)
    for i in range(NT):
        ztot = ztot + ztmp_v[pl.ds(i * 16, 16)]
    invv = jnp.full((16,), 1.0 / H, jnp.float32) / ztot
    for hh in range(2):
        h = 2 * c + hh
        inv = invv[hh]

        def scale(i, _, hh=hh, inv=inv):
            sl = pl.ds(hh * EPT + i * 16, 16)
            exp_v[sl] = exp_v[sl] * inv
            return 0

        lax.fori_loop(0, EPT // 16, scale, 0)
        pltpu.sync_copy(exp_v.at[pl.ds(hh * EPT, EPT)],
                        att_hbm.at[pl.ds(h * E + base, EPT)])


def _softmax(s2d, src, dst):
    mesh = plsc.VectorSubcoreMesh(core_axis_name="c", subcore_axis_name="s")
    return pl.kernel(
        _softmax_body,
        out_type=jax.ShapeDtypeStruct((P,), jnp.float32),
        mesh=mesh,
        compiler_params=pltpu.CompilerParams(needs_layout_passes=False),
        scratch_types=[
            pltpu.VMEM((EPT,), jnp.int32),
            pltpu.VMEM((EPT,), jnp.int32),
            pltpu.VMEM((N * 2 * H,), jnp.float32),
            pltpu.VMEM((2 * EPT,), jnp.float32),
            pltpu.VMEM((16,), jnp.float32),
            pltpu.VMEM((NT * 16,), jnp.float32),
            pltpu.VMEM_SHARED((NT * 16,), jnp.float32),
        ],
    )(s2d, src, dst)


def _agg_body(xsplit, att_f, src_hbm, dst_hbm, hsplit, out_hbm,
              att_v, dst_v, gix_full, six_full, rows_v, acc_sh):
    c = lax.axis_index("c")
    t = lax.axis_index("s")
    pltpu.sync_copy(hsplit.at[c, pl.ds(t * RSTART, RLEN), :],
                    acc_sh.at[pl.ds(t * RSTART, RLEN), :])
    plsc.subcore_barrier()

    xtab = xsplit.at[c]
    h = t // (NT // H)
    hoff = h * N
    pair0 = t * PPT
    ebase = (t % (NT // H)) * PPT

    def chunk(j, _):
        pltpu.sync_copy(dst_hbm.at[pl.ds(ebase + j * K, K)], dst_v)
        pltpu.sync_copy(src_hbm.at[pl.ds(ebase + j * K, K)], six_full)
        pltpu.sync_copy(att_f.at[pl.ds(pair0 + j * K, K)], att_v)
        for g in range(K // 16):
            sl = pl.ds(g * 16, 16)
            gix_full[sl] = dst_v[sl] + hoff
        pltpu.sync_copy(xtab.at[gix_full], rows_v)

        for g in range(K // 16):
            wv = att_v[pl.ds(g * 16, 16)]
            for rr in range(16):
                w = wv[rr]
                r = g * 16 + rr
                for v in range(HD // 16):
                    sl = pl.ds(v * 16, 16)
                    rows_v[r, sl] = rows_v[r, sl] * w

        pltpu.sync_copy(rows_v, acc_sh.at[six_full], add=True)
        return 0

    lax.fori_loop(0, PPT // K, chunk, 0)

    plsc.subcore_barrier()
    pltpu.sync_copy(acc_sh.at[pl.ds(t * RSTART, RLEN), :],
                    out_hbm.at[c, pl.ds(t * RSTART, RLEN), :])


def _aggregate(xflat, att_f, src, dst, hsplit):
    mesh = plsc.VectorSubcoreMesh(core_axis_name="c", subcore_axis_name="s")
    return pl.kernel(
        _agg_body,
        out_type=jax.ShapeDtypeStruct((2, N, HD), jnp.float32),
        mesh=mesh,
        compiler_params=pltpu.CompilerParams(needs_layout_passes=False),
        scratch_types=[
            pltpu.VMEM((K,), jnp.float32),
            pltpu.VMEM((K,), jnp.int32),
            pltpu.VMEM((K,), jnp.int32),
            pltpu.VMEM((K,), jnp.int32),
            pltpu.VMEM((K, HD), jnp.float32),
            pltpu.VMEM_SHARED((N, HD), jnp.float32),
        ],
    )(xflat, att_f, src, dst, hsplit)


@jax.jit
def kernel(input_h, edges, W, a):
    src = edges[:, 0].astype(jnp.int32)
    dst = edges[:, 2].astype(jnp.int32)

    xsplit, hsplit, smat = _dense(input_h, W, a)
    s2d = smat.reshape(-1)
    xflat = xsplit.reshape(2, H * N, HD)

    s2d, src, dst, xflat, hsplit = jax.lax.optimization_barrier(
        (s2d, src, dst, xflat, hsplit))
    att_f = _softmax(s2d, src, dst)
    att_f = jax.lax.optimization_barrier(att_f)
    hout = _aggregate(xflat, att_f, src, dst, hsplit)
    return jnp.concatenate([hout[0], hout[1]], axis=1)
